# baseline (speedup 1.0000x reference)
import functools
import math

import jax
import jax.numpy as jnp
from jax import lax
from jax.experimental import pallas as pl
from jax.experimental.pallas import tpu as pltpu
from jax.experimental.pallas import tpu_sc as plsc

VOCAB = 100
EMBED = 512
B = 1024
S = 200
LANES = 16
NUM_CORES = 2
NUM_SUBCORES = 16
NW = NUM_CORES * NUM_SUBCORES
BPW = B // NW
NB = 8
SB = 5
NBT = BPW // NB
NST = S // SB
NT = NST * NBT
PE_ROWS = 40
TILES_PER_PE = (PE_ROWS // SB) * NBT
GROUPS = EMBED // LANES
HALF = GROUPS // 4
SCALE = math.sqrt(EMBED)


def _body(x_hbm, table_hbm, pe_hbm, out_hbm,
          idx_all, table_v, pe_v, o0, o1, so0, so1):
    cid = lax.axis_index("c")
    sid = lax.axis_index("s")
    wid = sid * NUM_CORES + cid
    b0 = wid * BPW

    o = (o0, o1)
    so = (so0, so1)

    pltpu.sync_copy(x_hbm.at[pl.ds(pl.multiple_of(b0 * S, 8), BPW * S)],
                    idx_all.at[pl.ds(0, BPW * S)])
    pltpu.sync_copy(table_hbm, table_v)
    pltpu.sync_copy(pe_hbm.at[pl.ds(0, PE_ROWS), :], pe_v)

    @plsc.parallel_loop(0, VOCAB * EMBED, LANES)
    def _(i):
        sl = pl.ds(i, LANES)
        table_v[sl] = table_v[sl] * SCALE

    def pair(i, _):
        for p in (0, 1):
            t = 2 * i + p
            stile = t // NBT
            btile = lax.rem(t, NBT)
            s0 = stile * SB
            bt0 = btile * NB

            @pl.when(jnp.logical_and(lax.rem(t, TILES_PER_PE) == 0, t > 0))
            def _():
                off = pl.multiple_of((t // TILES_PER_PE) * PE_ROWS, 8)
                pltpu.sync_copy(pe_hbm.at[pl.ds(off, PE_ROWS), :], pe_v)

            @pl.when(t >= 2)
            def _():
                for _bi in range(NB):
                    pltpu.make_async_copy(
                        o[p].at[0], out_hbm.at[pl.ds(0, SB * EMBED)],
                        so[p]).wait()

            prow = lax.rem(stile, PE_ROWS // SB) * SB
            for s_local in range(SB):
                for jh in range(4):
                    pe_reg = [
                        pe_v[prow + s_local,
                             pl.ds((jh * HALF + j) * LANES, LANES)]
                        for j in range(HALF)
                    ]
                    i_base = bt0 * S + s0 + s_local

                    @plsc.parallel_loop(0, NB)
                    def _(bi):
                        iv = idx_all[pl.ds(i_base + bi * S, LANES)]
                        row = pl.multiple_of(iv[0] * EMBED, LANES)
                        for j in range(HALF):
                            g = jh * HALF + j
                            o[p][bi, pl.ds((s_local * GROUPS + g) * LANES,
                                           LANES)] = (
                                table_v[pl.ds(row + g * LANES, LANES)]
                                + pe_reg[j])

            for bi in range(NB):
                off = pl.multiple_of(
                    (b0 + bt0 + bi) * S * EMBED + s0 * EMBED, 8)
                pltpu.async_copy(
                    o[p].at[bi], out_hbm.at[pl.ds(off, SB * EMBED)], so[p])
        return 0

    lax.fori_loop(0, NT // 2, pair, 0)

    for p in (0, 1):
        for _bi in range(NB):
            pltpu.make_async_copy(
                o[p].at[0], out_hbm.at[pl.ds(0, SB * EMBED)], so[p]).wait()


@jax.jit
def kernel(x, table, pe):
    run = functools.partial(
        pl.kernel,
        out_type=jax.ShapeDtypeStruct((B * S * EMBED,), jnp.float32),
        mesh=plsc.VectorSubcoreMesh(core_axis_name="c", subcore_axis_name="s"),
        scratch_types=[
            pltpu.VMEM((BPW * S + LANES,), jnp.int32),
            pltpu.VMEM((VOCAB * EMBED,), jnp.float32),
            pltpu.VMEM((PE_ROWS, EMBED), jnp.float32),
            pltpu.VMEM((NB, SB * EMBED), jnp.float32),
            pltpu.VMEM((NB, SB * EMBED), jnp.float32),
            pltpu.SemaphoreType.DMA,
            pltpu.SemaphoreType.DMA,
        ],
    )(_body)
    return run(x.reshape(B * S), table.reshape(VOCAB * EMBED),
               pe).reshape(B, S, EMBED)


# speedup vs baseline: 3.1488x; 3.1488x over previous
import functools
import math

import jax
import jax.numpy as jnp
from jax import lax
from jax.experimental import pallas as pl
from jax.experimental.pallas import tpu as pltpu
from jax.experimental.pallas import tpu_sc as plsc

VOCAB = 100
EMBED = 512
B = 1024
S = 200
LANES = 16
NUM_CORES = 2
NUM_SUBCORES = 16
NW = NUM_CORES * NUM_SUBCORES
BPW = B // NW
C = 40
NSC = S // C
NT = NSC * BPW
GROUPS = EMBED // LANES
SCALE = math.sqrt(EMBED)


def _body(x_hbm, table_hbm, pe_hbm, out_hbm,
          idx_all, table_v, pe_sh, o0, o1, so0, so1, sf0, sf1):
    cid = lax.axis_index("c")
    sid = lax.axis_index("s")
    wid = sid * NUM_CORES + cid
    b0 = wid * BPW

    o = (o0, o1)
    so = (so0, so1)
    sf = (sf0, sf1)

    def s_base_of(t):
        return pl.multiple_of((t // BPW) * C, 8)

    def issue_prefill(t, p):
        pltpu.async_copy(pe_sh.at[pl.ds(s_base_of(t), C), :], o[p], sf[p])

    pltpu.sync_copy(x_hbm.at[pl.ds(pl.multiple_of(b0 * S, 8), BPW * S)],
                    idx_all.at[pl.ds(0, BPW * S)])
    pltpu.sync_copy(table_hbm, table_v)

    @pl.when(sid == 0)
    def _():
        pltpu.sync_copy(pe_hbm.at[pl.ds(0, S), :], pe_sh)

    @plsc.parallel_loop(0, VOCAB * EMBED, LANES)
    def _(i):
        sl = pl.ds(i, LANES)
        table_v[sl] = table_v[sl] * SCALE

    plsc.subcore_barrier()
    issue_prefill(0, 0)

    def pair(i, _):
        for p in (0, 1):
            t = 2 * i + p
            s_base = s_base_of(t)
            bi = lax.rem(t, BPW)
            b = b0 + bi

            pltpu.make_async_copy(
                pe_sh.at[pl.ds(0, C), :], o[p], sf[p]).wait()

            i_base = bi * S + s_base

            @plsc.parallel_loop(0, C)
            def _(r):
                iv = idx_all[pl.ds(i_base + r, LANES)]
                row = pl.multiple_of(iv[0] * EMBED, LANES)
                for j in range(GROUPS):
                    plsc.addupdate(
                        o[p].at[r, pl.ds(j * LANES, LANES)],
                        table_v[pl.ds(row + j * LANES, LANES)])

            pltpu.async_copy(o[p], out_hbm.at[b, pl.ds(s_base, C), :], so[p])

            @pl.when(t >= 1)
            def _():
                pltpu.make_async_copy(
                    o[1 - p], out_hbm.at[0, pl.ds(0, C), :], so[1 - p]).wait()

            @pl.when(t + 1 < NT)
            def _():
                issue_prefill(t + 1, 1 - p)
        return 0

    lax.fori_loop(0, NT // 2, pair, 0)

    pltpu.make_async_copy(
        o[(NT - 1) % 2], out_hbm.at[0, pl.ds(0, C), :],
        so[(NT - 1) % 2]).wait()


@jax.jit
def kernel(x, table, pe):
    run = functools.partial(
        pl.kernel,
        out_type=jax.ShapeDtypeStruct((B, S, EMBED), jnp.float32),
        mesh=plsc.VectorSubcoreMesh(core_axis_name="c", subcore_axis_name="s"),
        scratch_types=[
            pltpu.VMEM((BPW * S + LANES,), jnp.int32),
            pltpu.VMEM((VOCAB * EMBED,), jnp.float32),
            pltpu.VMEM_SHARED((S, EMBED), jnp.float32),
            pltpu.VMEM((C, EMBED), jnp.float32),
            pltpu.VMEM((C, EMBED), jnp.float32),
            pltpu.SemaphoreType.DMA,
            pltpu.SemaphoreType.DMA,
            pltpu.SemaphoreType.DMA,
            pltpu.SemaphoreType.DMA,
        ],
    )(_body)
    return run(x.reshape(B * S), table.reshape(VOCAB * EMBED), pe)
